# Initial kernel scaffold; baseline (speedup 1.0000x reference)
import jax
import jax.numpy as jnp
from jax.experimental import pallas as pl


def kernel(x_movie, x_director, x_actor, src_md, dst_md, src_dm, dst_dm, src_ma, dst_ma, src_am, dst_am, Wp_movie, bp_movie, Wp_director, bp_director, Wp_actor, bp_actor, a_src_md, a_dst_md, a_src_dm, a_dst_dm, a_src_ma, a_dst_ma, a_src_am, a_dst_am, kW, kb, q, lin_W, lin_b):
    raise NotImplementedError("write your pallas kernel here")


# speedup vs baseline: 9.5361x; 9.5361x over previous
import functools

import jax
import jax.numpy as jnp
from jax import lax
from jax.experimental import pallas as pl
from jax.experimental.pallas import tpu as pltpu
from jax.experimental.pallas import tpu_sc as plsc

N_MOVIE, N_DIRECTOR, N_ACTOR = 50000, 10000, 30000
D_IN = 128
HIDDEN, HEADS, OUT = 128, 8, 3
D = HIDDEN // HEADS

BLK = 1000
F32 = jnp.float32
I32 = jnp.int32


def _proj_body(x_ref, w_ref, b_ref, a_ref, h_ref, al_ref, mx_ref):
    h = jnp.dot(x_ref[...], w_ref[...], preferred_element_type=F32) + b_ref[...]
    h_ref[...] = h
    al = jnp.dot(h, a_ref[...], preferred_element_type=F32)
    al_ref[...] = al
    bm = jnp.max(al, axis=0, keepdims=True)

    @pl.when(pl.program_id(0) == 0)
    def _():
        mx_ref[...] = bm

    @pl.when(pl.program_id(0) > 0)
    def _():
        mx_ref[...] = jnp.maximum(mx_ref[...], bm)


def _project(x, W, b, A, n):
    return pl.pallas_call(
        _proj_body,
        grid=(n // BLK,),
        in_specs=[pl.BlockSpec((BLK, 128), lambda i: (i, 0)),
                  pl.BlockSpec((128, 128), lambda i: (0, 0)),
                  pl.BlockSpec((1, 128), lambda i: (0, 0)),
                  pl.BlockSpec((128, 128), lambda i: (0, 0))],
        out_specs=[pl.BlockSpec((BLK, 128), lambda i: (i, 0)),
                   pl.BlockSpec((BLK, 128), lambda i: (i, 0)),
                   pl.BlockSpec((1, 128), lambda i: (0, 0))],
        out_shape=[jax.ShapeDtypeStruct((n, 128), F32),
                   jax.ShapeDtypeStruct((n, 128), F32),
                   jax.ShapeDtypeStruct((1, 128), F32)],
    )(x, W, b.reshape(1, 128), A)


def _tansum_body(o1_ref, o2_ref, kw_ref, kb_ref, t_ref):
    t1 = jnp.tanh(jnp.dot(o1_ref[...], kw_ref[...], preferred_element_type=F32)
                  + kb_ref[...])
    t2 = jnp.tanh(jnp.dot(o2_ref[...], kw_ref[...], preferred_element_type=F32)
                  + kb_ref[...])
    blk = jnp.concatenate([t1.sum(0, keepdims=True), t2.sum(0, keepdims=True)],
                          axis=0)

    @pl.when(pl.program_id(0) == 0)
    def _():
        t_ref[...] = blk

    @pl.when(pl.program_id(0) > 0)
    def _():
        t_ref[...] = t_ref[...] + blk


def _tansum(o1, o2, kW, kb):
    return pl.pallas_call(
        _tansum_body,
        grid=(N_MOVIE // BLK,),
        in_specs=[pl.BlockSpec((BLK, 128), lambda i: (i, 0)),
                  pl.BlockSpec((BLK, 128), lambda i: (i, 0)),
                  pl.BlockSpec((128, 128), lambda i: (0, 0)),
                  pl.BlockSpec((1, 128), lambda i: (0, 0))],
        out_specs=pl.BlockSpec((2, 128), lambda i: (0, 0)),
        out_shape=jax.ShapeDtypeStruct((2, 128), F32),
    )(o1, o2, kW, kb.reshape(1, 128))


def _combine_body(o1_ref, o2_ref, t_ref, q_ref, lw_ref, lb_ref, mo_ref, ls_ref):
    qv = q_ref[...]
    t = t_ref[...]
    s1 = jnp.sum(qv[0] * t[0]) / N_MOVIE
    s2 = jnp.sum(qv[0] * t[1]) / N_MOVIE
    mx = jnp.maximum(s1, s2)
    e1 = jnp.exp(s1 - mx)
    e2 = jnp.exp(s2 - mx)
    a1 = e1 / (e1 + e2)
    a2 = e2 / (e1 + e2)
    mo = a1 * o1_ref[...] + a2 * o2_ref[...]
    mo_ref[...] = mo
    lg = jnp.dot(mo, lw_ref[...], preferred_element_type=F32) + lb_ref[...]
    col = lax.broadcasted_iota(I32, (BLK, 128), 1)
    msk = col < OUT
    m3 = jnp.max(jnp.where(msk, lg, -jnp.inf), axis=1, keepdims=True)
    ex = jnp.where(msk, jnp.exp(lg - m3), 0.0)
    ssum = jnp.sum(ex, axis=1, keepdims=True)
    ls_ref[...] = jnp.where(msk, lg - m3 - jnp.log(ssum), 0.0)


def _combine(o1, o2, t, q, lin_W, lin_b):
    lw = jnp.zeros((128, 128), F32).at[:, :OUT].set(lin_W)
    lb = jnp.zeros((1, 128), F32).at[0, :OUT].set(lin_b)
    return pl.pallas_call(
        _combine_body,
        grid=(N_MOVIE // BLK,),
        in_specs=[pl.BlockSpec((BLK, 128), lambda i: (i, 0)),
                  pl.BlockSpec((BLK, 128), lambda i: (i, 0)),
                  pl.BlockSpec((2, 128), lambda i: (0, 0)),
                  pl.BlockSpec((1, 128), lambda i: (0, 0)),
                  pl.BlockSpec((128, 128), lambda i: (0, 0)),
                  pl.BlockSpec((1, 128), lambda i: (0, 0))],
        out_specs=[pl.BlockSpec((BLK, 128), lambda i: (i, 0)),
                   pl.BlockSpec((BLK, 128), lambda i: (i, 0))],
        out_shape=[jax.ShapeDtypeStruct((N_MOVIE, 128), F32),
                   jax.ShapeDtypeStruct((N_MOVIE, 128), F32)],
    )(o1, o2, t, q.reshape(1, 128), lw, lb)


def _make_edge_kernel(E, NB, K, HG, n_src, n_dst):
    Lt = NB * K
    E32 = 16 * Lt
    NG = 8 // HG
    NPH = NG // 2
    W = 16 * HG + 16
    RT = n_dst // 16
    DB = 125
    NDB = RT // DB
    mesh = plsc.VectorSubcoreMesh(core_axis_name="c", subcore_axis_name="s")

    @functools.partial(
        pl.kernel,
        out_type=jax.ShapeDtypeStruct((NG, n_dst, 16 * HG), F32),
        mesh=mesh,
        compiler_params=pltpu.CompilerParams(use_tc_tiling_on_sc=False),
        scratch_types=dict(
            bsrcE=pltpu.VMEM((K,), I32),
            bdstE=pltpu.VMEM((K,), I32),
            bsrc=pltpu.VMEM((K,), I32),
            ga=pltpu.VMEM((K, 16), F32),
            gd=pltpu.VMEM((K, 16), F32),
            gh=pltpu.VMEM((K, 16 * HG), F32),
            scb=pltpu.VMEM((K, W), F32),
            mvb=pltpu.VMEM((16,), F32),
            dra=pltpu.VMEM((DB, W), F32),
            drb=pltpu.VMEM((DB, 16 * HG), F32),
            acc=pltpu.VMEM_SHARED((n_dst, W), F32),
            sem=pltpu.SemaphoreType.DMA,
        ),
    )
    def edge_kernel(hT_hbm, asrc_hbm, adst_hbm, src_hbm, dst_hbm, mshift_hbm,
                    out_hbm, bsrcE, bdstE, bsrc, ga, gd, gh, scb, mvb,
                    dra, drb, acc, sem):
        c = lax.axis_index("c")
        s = lax.axis_index("s")
        pltpu.sync_copy(mshift_hbm, mvb)
        ecut = E - s * Lt
        zrow = jnp.zeros((16,), F32)

        for p in range(NPH):
            gflat = (c * NPH + p) * n_src

            def zbody(r, _):
                for w16 in range(W // 16):
                    dra[r, pl.ds(w16 * 16, 16)] = zrow
                return 0
            lax.fori_loop(0, DB, zbody, 0)
            for blk in range(NDB):
                pltpu.sync_copy(dra, acc.at[pl.ds(s * RT + blk * DB, DB)])
            plsc.subcore_barrier()

            def bbody(b, _):
                off = b * K
                pltpu.sync_copy(src_hbm.at[pl.ds(s * Lt + off, K)], bsrcE)
                pltpu.sync_copy(dst_hbm.at[pl.ds(s * Lt + off, K)], bdstE)
                for j in range(K // 16):
                    sl = pl.ds(j * 16, 16)
                    bsrc[sl] = bsrcE[sl] + gflat
                pltpu.async_copy(asrc_hbm.at[bsrcE], ga, sem).wait()
                pltpu.async_copy(adst_hbm.at[bdstE], gd, sem).wait()
                pltpu.async_copy(hT_hbm.at[bsrc], gh, sem).wait()

                def ebody(e, _2):
                    a = ga[e] + gd[e]
                    a = jnp.where(a > 0, a, 0.2 * a)
                    ex = jnp.exp(a - mvb[...])
                    ex = ex * ((off + e) < ecut).astype(F32)
                    scb[e, pl.ds(16 * HG, 16)] = ex
                    for j in range(HG):
                        w = jnp.where(c == 0, ex[p * HG + j],
                                      ex[NPH * HG + p * HG + j])
                        scb[e, pl.ds(16 * j, 16)] = gh[e, pl.ds(16 * j, 16)] * w
                    return 0
                lax.fori_loop(0, K, ebody, 0)
                pltpu.sync_copy(scb, acc.at[bdstE], add=True)
                return 0
            lax.fori_loop(0, NB, bbody, 0)
            plsc.subcore_barrier()

            for blk in range(NDB):
                row0 = s * RT + blk * DB
                pltpu.sync_copy(acc.at[pl.ds(row0, DB)], dra)

                def dbody(r, _):
                    exsec = dra[r, pl.ds(16 * HG, 16)]
                    for j in range(HG):
                        dn = jnp.where(c == 0, exsec[p * HG + j],
                                       exsec[NPH * HG + p * HG + j])
                        v = dra[r, pl.ds(16 * j, 16)] / (dn + 1e-16)
                        drb[r, pl.ds(16 * j, 16)] = jnp.maximum(v, 0.0)
                    return 0
                lax.fori_loop(0, DB, dbody, 0)
                pltpu.sync_copy(drb, out_hbm.at[c * NPH + p, pl.ds(row0, DB)])
            plsc.subcore_barrier()

    return edge_kernel


_edge_md = _make_edge_kernel(E=50000, NB=13, K=256, HG=4, n_src=50000, n_dst=10000)
_edge_dm = _make_edge_kernel(E=50000, NB=26, K=128, HG=1, n_src=10000, n_dst=50000)
_edge_ma = _make_edge_kernel(E=150000, NB=74, K=128, HG=2, n_src=50000, n_dst=30000)
_edge_am = _make_edge_kernel(E=150000, NB=74, K=128, HG=1, n_src=30000, n_dst=50000)


def _amat(blocks):
    eye8 = jnp.eye(8, dtype=F32)
    cols = [jnp.einsum('hd,hk->hdk', a, eye8).reshape(128, 8) for a in blocks]
    A = jnp.concatenate(cols, axis=1)
    return jnp.pad(A, ((0, 0), (0, 128 - A.shape[1])))


def _pad_edges(src, dst, E32):
    E = src.shape[0]
    src_p = jnp.concatenate([src, jnp.zeros((E32 - E,), I32)])
    dst_p = jnp.concatenate([dst, jnp.zeros((E32 - E,), I32)])
    return src_p, dst_p


def _mshift(cmax_src, c0s, cmax_dst, c0d):
    mv = cmax_src[0, c0s:c0s + 8] + cmax_dst[0, c0d:c0d + 8]
    mv = jnp.where(mv > 0, mv, 0.2 * mv)
    return jnp.concatenate([mv, jnp.zeros((8,), F32)])


def _headmajor(h, HG, n):
    NG = 8 // HG
    return jnp.transpose(h.reshape(n, NG, 16 * HG), (1, 0, 2)).reshape(
        NG * n, 16 * HG)


def _regroup(o, n):
    return jnp.transpose(o, (1, 0, 2)).reshape(n, 128)


def kernel(x_movie, x_director, x_actor, src_md, dst_md, src_dm, dst_dm,
           src_ma, dst_ma, src_am, dst_am, Wp_movie, bp_movie, Wp_director,
           bp_director, Wp_actor, bp_actor, a_src_md, a_dst_md, a_src_dm,
           a_dst_dm, a_src_ma, a_dst_ma, a_src_am, a_dst_am, kW, kb, q,
           lin_W, lin_b):
    A_m = _amat([a_src_md, a_src_ma, a_dst_dm, a_dst_am])
    A_d = _amat([a_dst_md, a_src_dm])
    A_a = _amat([a_dst_ma, a_src_am])

    h_m, al_m, mx_m = _project(x_movie, Wp_movie, bp_movie, A_m, N_MOVIE)
    h_d, al_d, mx_d = _project(x_director, Wp_director, bp_director, A_d,
                               N_DIRECTOR)
    h_a, al_a, mx_a = _project(x_actor, Wp_actor, bp_actor, A_a, N_ACTOR)

    s_md, d_md = _pad_edges(src_md, dst_md, 53248)
    s_dm, d_dm = _pad_edges(src_dm, dst_dm, 53248)
    s_ma, d_ma = _pad_edges(src_ma, dst_ma, 151552)
    s_am, d_am = _pad_edges(src_am, dst_am, 151552)

    out_md = _edge_md(_headmajor(h_m, 4, N_MOVIE), al_m[:, 0:16],
                      al_d[:, 0:16], s_md, d_md, _mshift(mx_m, 0, mx_d, 0))
    out_dm = _edge_dm(_headmajor(h_d, 1, N_DIRECTOR), al_d[:, 8:24],
                      al_m[:, 16:32], s_dm, d_dm, _mshift(mx_d, 8, mx_m, 16))
    out_ma = _edge_ma(_headmajor(h_m, 2, N_MOVIE), al_m[:, 8:24],
                      al_a[:, 0:16], s_ma, d_ma, _mshift(mx_m, 8, mx_a, 0))
    out_am = _edge_am(_headmajor(h_a, 1, N_ACTOR), al_a[:, 8:24],
                      al_m[:, 24:40], s_am, d_am, _mshift(mx_a, 8, mx_m, 24))

    o_dm = _regroup(out_dm, N_MOVIE)
    o_am = _regroup(out_am, N_MOVIE)
    director_out = _regroup(out_md, N_DIRECTOR)
    actor_out = _regroup(out_ma, N_ACTOR)

    t = _tansum(o_dm, o_am, kW, kb)
    movie_out, lsm = _combine(o_dm, o_am, t, q, lin_W, lin_b)
    return (lsm[:, :OUT], movie_out, director_out, actor_out)


# speedup vs baseline: 14.6323x; 1.5344x over previous
import functools

import jax
import jax.numpy as jnp
from jax import lax
from jax.experimental import pallas as pl
from jax.experimental.pallas import tpu as pltpu
from jax.experimental.pallas import tpu_sc as plsc

N_MOVIE, N_DIRECTOR, N_ACTOR = 50000, 10000, 30000
D_IN = 128
HIDDEN, HEADS, OUT = 128, 8, 3
D = HIDDEN // HEADS

BLK = 1000
F32 = jnp.float32
I32 = jnp.int32


def _proj_body(x_ref, w_ref, b_ref, a_ref, h_ref, al_ref, mx_ref):
    h = jnp.dot(x_ref[...], w_ref[...], preferred_element_type=F32) + b_ref[...]
    h_ref[...] = h
    al = jnp.dot(h, a_ref[...], preferred_element_type=F32)
    al_ref[...] = al
    bm = jnp.max(al, axis=0, keepdims=True)

    @pl.when(pl.program_id(0) == 0)
    def _():
        mx_ref[...] = bm

    @pl.when(pl.program_id(0) > 0)
    def _():
        mx_ref[...] = jnp.maximum(mx_ref[...], bm)


def _project(x, W, b, A, n):
    return pl.pallas_call(
        _proj_body,
        grid=(n // BLK,),
        in_specs=[pl.BlockSpec((BLK, 128), lambda i: (i, 0)),
                  pl.BlockSpec((128, 128), lambda i: (0, 0)),
                  pl.BlockSpec((1, 128), lambda i: (0, 0)),
                  pl.BlockSpec((128, 128), lambda i: (0, 0))],
        out_specs=[pl.BlockSpec((BLK, 128), lambda i: (i, 0)),
                   pl.BlockSpec((BLK, 128), lambda i: (i, 0)),
                   pl.BlockSpec((1, 128), lambda i: (0, 0))],
        out_shape=[jax.ShapeDtypeStruct((n, 128), F32),
                   jax.ShapeDtypeStruct((n, 128), F32),
                   jax.ShapeDtypeStruct((1, 128), F32)],
    )(x, W, b.reshape(1, 128), A)


def _tansum_body(o1_ref, o2_ref, kw_ref, kb_ref, t_ref):
    t1 = jnp.tanh(jnp.dot(o1_ref[...], kw_ref[...], preferred_element_type=F32)
                  + kb_ref[...])
    t2 = jnp.tanh(jnp.dot(o2_ref[...], kw_ref[...], preferred_element_type=F32)
                  + kb_ref[...])
    blk = jnp.concatenate([t1.sum(0, keepdims=True), t2.sum(0, keepdims=True)],
                          axis=0)

    @pl.when(pl.program_id(0) == 0)
    def _():
        t_ref[...] = blk

    @pl.when(pl.program_id(0) > 0)
    def _():
        t_ref[...] = t_ref[...] + blk


def _tansum(o1, o2, kW, kb):
    return pl.pallas_call(
        _tansum_body,
        grid=(N_MOVIE // BLK,),
        in_specs=[pl.BlockSpec((BLK, 128), lambda i: (i, 0)),
                  pl.BlockSpec((BLK, 128), lambda i: (i, 0)),
                  pl.BlockSpec((128, 128), lambda i: (0, 0)),
                  pl.BlockSpec((1, 128), lambda i: (0, 0))],
        out_specs=pl.BlockSpec((2, 128), lambda i: (0, 0)),
        out_shape=jax.ShapeDtypeStruct((2, 128), F32),
    )(o1, o2, kW, kb.reshape(1, 128))


def _combine_body(o1_ref, o2_ref, t_ref, q_ref, lw_ref, lb_ref, mo_ref, ls_ref):
    qv = q_ref[...]
    t = t_ref[...]
    s1 = jnp.sum(qv[0] * t[0]) / N_MOVIE
    s2 = jnp.sum(qv[0] * t[1]) / N_MOVIE
    mx = jnp.maximum(s1, s2)
    e1 = jnp.exp(s1 - mx)
    e2 = jnp.exp(s2 - mx)
    a1 = e1 / (e1 + e2)
    a2 = e2 / (e1 + e2)
    mo = a1 * o1_ref[...] + a2 * o2_ref[...]
    mo_ref[...] = mo
    lg = jnp.dot(mo, lw_ref[...], preferred_element_type=F32) + lb_ref[...]
    col = lax.broadcasted_iota(I32, (BLK, 128), 1)
    msk = col < OUT
    m3 = jnp.max(jnp.where(msk, lg, -jnp.inf), axis=1, keepdims=True)
    ex = jnp.where(msk, jnp.exp(lg - m3), 0.0)
    ssum = jnp.sum(ex, axis=1, keepdims=True)
    ls_ref[...] = jnp.where(msk, lg - m3 - jnp.log(ssum), 0.0)


def _combine(o1, o2, t, q, lin_W, lin_b):
    lw = jnp.zeros((128, 128), F32).at[:, :OUT].set(lin_W)
    lb = jnp.zeros((1, 128), F32).at[0, :OUT].set(lin_b)
    return pl.pallas_call(
        _combine_body,
        grid=(N_MOVIE // BLK,),
        in_specs=[pl.BlockSpec((BLK, 128), lambda i: (i, 0)),
                  pl.BlockSpec((BLK, 128), lambda i: (i, 0)),
                  pl.BlockSpec((2, 128), lambda i: (0, 0)),
                  pl.BlockSpec((1, 128), lambda i: (0, 0)),
                  pl.BlockSpec((128, 128), lambda i: (0, 0)),
                  pl.BlockSpec((1, 128), lambda i: (0, 0))],
        out_specs=[pl.BlockSpec((BLK, 128), lambda i: (i, 0)),
                   pl.BlockSpec((BLK, 128), lambda i: (i, 0))],
        out_shape=[jax.ShapeDtypeStruct((N_MOVIE, 128), F32),
                   jax.ShapeDtypeStruct((N_MOVIE, 128), F32)],
    )(o1, o2, t, q.reshape(1, 128), lw, lb)


def _make_edge_kernel(E, NB, K, HG, n_src, n_dst, DB):
    Lt = NB * K
    E32 = 16 * Lt
    NG = 8 // HG
    NPH = NG // 2
    W = 16 * HG + 16
    RT = n_dst // 16
    NDB = RT // DB
    assert NB % 2 == 0 and NB >= 4 and RT % DB == 0
    mesh = plsc.VectorSubcoreMesh(core_axis_name="c", subcore_axis_name="s")

    scr = dict(
        mvb=pltpu.VMEM((16,), F32),
        dra=pltpu.VMEM((DB, W), F32),
        drb=pltpu.VMEM((DB, 16 * HG), F32),
        acc=pltpu.VMEM_SHARED((n_dst, W), F32),
    )
    for i in range(2):
        scr[f"bsrcE{i}"] = pltpu.VMEM((K,), I32)
        scr[f"bdstE{i}"] = pltpu.VMEM((K,), I32)
        scr[f"bsrc{i}"] = pltpu.VMEM((K,), I32)
        scr[f"bdst{i}"] = pltpu.VMEM((K,), I32)
        scr[f"ga{i}"] = pltpu.VMEM((K, 16), F32)
        scr[f"gd{i}"] = pltpu.VMEM((K, 16), F32)
        scr[f"gh{i}"] = pltpu.VMEM((K, 16 * HG), F32)
        scr[f"scb{i}"] = pltpu.VMEM((K, W), F32)
        scr[f"seme{i}"] = pltpu.SemaphoreType.DMA
        scr[f"semg{i}"] = pltpu.SemaphoreType.DMA
        scr[f"sems{i}"] = pltpu.SemaphoreType.DMA

    @functools.partial(
        pl.kernel,
        out_type=jax.ShapeDtypeStruct((NG, n_dst, 16 * HG), F32),
        mesh=mesh,
        compiler_params=pltpu.CompilerParams(use_tc_tiling_on_sc=False),
        scratch_types=scr,
    )
    def edge_kernel(hT_hbm, asrc_hbm, adst_hbm, src_hbm, dst_hbm, mshift_hbm,
                    out_hbm, mvb, dra, drb, acc,
                    bsrcE0, bdstE0, bsrc0, bdst0, ga0, gd0, gh0, scb0,
                    seme0, semg0, sems0,
                    bsrcE1, bdstE1, bsrc1, bdst1, ga1, gd1, gh1, scb1,
                    seme1, semg1, sems1):
        bsrcE = (bsrcE0, bsrcE1)
        bdstE = (bdstE0, bdstE1)
        bsrc = (bsrc0, bsrc1)
        bdst = (bdst0, bdst1)
        ga = (ga0, ga1)
        gd = (gd0, gd1)
        gh = (gh0, gh1)
        scb = (scb0, scb1)
        seme = (seme0, seme1)
        semg = (semg0, semg1)
        sems = (sems0, sems1)
        c = lax.axis_index("c")
        s = lax.axis_index("s")
        pltpu.sync_copy(mshift_hbm, mvb)
        ecut = E - s * Lt
        zrow = jnp.zeros((16,), F32)

        def issue_edges(b, par):
            off = s * Lt + b * K
            pltpu.async_copy(src_hbm.at[pl.ds(off, K)], bsrcE[par], seme[par])
            pltpu.async_copy(dst_hbm.at[pl.ds(off, K)], bdstE[par], seme[par])

        def wait_edges(par):
            pltpu.make_async_copy(src_hbm.at[pl.ds(0, K)], bsrcE[par],
                                  seme[par]).wait()
            pltpu.make_async_copy(dst_hbm.at[pl.ds(0, K)], bdstE[par],
                                  seme[par]).wait()

        def wait_scatter(par):
            pltpu.make_async_copy(scb[par], acc.at[bdst[par]],
                                  sems[par]).wait()

        for p in range(NPH):
            gflat = (c * NPH + p) * n_src

            def mk_idx(par):
                for j in range(K // 16):
                    sl = pl.ds(j * 16, 16)
                    bsrc[par][sl] = bsrcE[par][sl] + gflat
                    bdst[par][sl] = bdstE[par][sl]

            def issue_gathers(par):
                pltpu.async_copy(asrc_hbm.at[bsrcE[par]], ga[par], semg[par])
                pltpu.async_copy(adst_hbm.at[bdst[par]], gd[par], semg[par])
                pltpu.async_copy(hT_hbm.at[bsrc[par]], gh[par], semg[par])

            def wait_gathers(par):
                pltpu.make_async_copy(asrc_hbm.at[bsrcE[par]], ga[par],
                                      semg[par]).wait()
                pltpu.make_async_copy(adst_hbm.at[bdst[par]], gd[par],
                                      semg[par]).wait()
                pltpu.make_async_copy(hT_hbm.at[bsrc[par]], gh[par],
                                      semg[par]).wait()

            def compute(b, par):
                off = b * K

                def ebody(i, _2):
                    for u in range(2):
                        e = i * 2 + u
                        a = ga[par][e] + gd[par][e]
                        a = jnp.where(a > 0, a, 0.2 * a)
                        ex = jnp.exp(a - mvb[...])
                        ex = ex * ((off + e) < ecut).astype(F32)
                        scb[par][e, pl.ds(16 * HG, 16)] = ex
                        for j in range(HG):
                            w = jnp.where(c == 0, ex[p * HG + j],
                                          ex[NPH * HG + p * HG + j])
                            scb[par][e, pl.ds(16 * j, 16)] = (
                                gh[par][e, pl.ds(16 * j, 16)] * w)
                    return 0
                lax.fori_loop(0, K // 2, ebody, 0)

            def zbody(r, _):
                for w16 in range(W // 16):
                    dra[r, pl.ds(w16 * 16, 16)] = zrow
                return 0
            lax.fori_loop(0, DB, zbody, 0)
            for blk in range(NDB):
                pltpu.sync_copy(dra, acc.at[pl.ds(s * RT + blk * DB, DB)])
            plsc.subcore_barrier()

            issue_edges(0, 0)
            issue_edges(1, 1)
            wait_edges(0)
            mk_idx(0)
            issue_gathers(0)

            def pairbody(i, _):
                for par in range(2):
                    b = 2 * i + par
                    o = 1 - par

                    @pl.when(b + 1 < NB)
                    def _():
                        wait_edges(o)

                        @pl.when(b >= 1)
                        def _():
                            wait_scatter(o)
                        mk_idx(o)
                        issue_gathers(o)
                    wait_gathers(par)

                    @pl.when(b + 2 < NB)
                    def _():
                        issue_edges(b + 2, par)
                    compute(b, par)
                    pltpu.async_copy(scb[par], acc.at[bdst[par]], sems[par],
                                     add=True)
                return 0
            lax.fori_loop(0, NB // 2, pairbody, 0)
            wait_scatter(0)
            wait_scatter(1)
            plsc.subcore_barrier()

            for blk in range(NDB):
                row0 = s * RT + blk * DB
                pltpu.sync_copy(acc.at[pl.ds(row0, DB)], dra)

                def dbody(r, _):
                    exsec = dra[r, pl.ds(16 * HG, 16)]
                    for j in range(HG):
                        dn = jnp.where(c == 0, exsec[p * HG + j],
                                       exsec[NPH * HG + p * HG + j])
                        v = dra[r, pl.ds(16 * j, 16)] / (dn + 1e-16)
                        drb[r, pl.ds(16 * j, 16)] = jnp.maximum(v, 0.0)
                    return 0
                lax.fori_loop(0, DB, dbody, 0)
                pltpu.sync_copy(drb, out_hbm.at[c * NPH + p, pl.ds(row0, DB)])
            plsc.subcore_barrier()

    return edge_kernel


_edge_md = _make_edge_kernel(E=50000, NB=26, K=128, HG=4, n_src=50000,
                             n_dst=10000, DB=125)
_edge_dm = _make_edge_kernel(E=50000, NB=26, K=128, HG=1, n_src=10000,
                             n_dst=50000, DB=125)
_edge_ma = _make_edge_kernel(E=150000, NB=74, K=128, HG=2, n_src=50000,
                             n_dst=30000, DB=75)
_edge_am = _make_edge_kernel(E=150000, NB=74, K=128, HG=1, n_src=30000,
                             n_dst=50000, DB=125)


def _amat(blocks):
    eye8 = jnp.eye(8, dtype=F32)
    cols = [jnp.einsum('hd,hk->hdk', a, eye8).reshape(128, 8) for a in blocks]
    A = jnp.concatenate(cols, axis=1)
    return jnp.pad(A, ((0, 0), (0, 128 - A.shape[1])))


def _pad_edges(src, dst, E32):
    E = src.shape[0]
    src_p = jnp.concatenate([src, jnp.zeros((E32 - E,), I32)])
    dst_p = jnp.concatenate([dst, jnp.zeros((E32 - E,), I32)])
    return src_p, dst_p


def _mshift(cmax_src, c0s, cmax_dst, c0d):
    mv = cmax_src[0, c0s:c0s + 8] + cmax_dst[0, c0d:c0d + 8]
    mv = jnp.where(mv > 0, mv, 0.2 * mv)
    return jnp.concatenate([mv, jnp.zeros((8,), F32)])


def _headmajor(h, HG, n):
    NG = 8 // HG
    return jnp.transpose(h.reshape(n, NG, 16 * HG), (1, 0, 2)).reshape(
        NG * n, 16 * HG)


def _regroup(o, n):
    return jnp.transpose(o, (1, 0, 2)).reshape(n, 128)


def kernel(x_movie, x_director, x_actor, src_md, dst_md, src_dm, dst_dm,
           src_ma, dst_ma, src_am, dst_am, Wp_movie, bp_movie, Wp_director,
           bp_director, Wp_actor, bp_actor, a_src_md, a_dst_md, a_src_dm,
           a_dst_dm, a_src_ma, a_dst_ma, a_src_am, a_dst_am, kW, kb, q,
           lin_W, lin_b):
    A_m = _amat([a_src_md, a_src_ma, a_dst_dm, a_dst_am])
    A_d = _amat([a_dst_md, a_src_dm])
    A_a = _amat([a_dst_ma, a_src_am])

    h_m, al_m, mx_m = _project(x_movie, Wp_movie, bp_movie, A_m, N_MOVIE)
    h_d, al_d, mx_d = _project(x_director, Wp_director, bp_director, A_d,
                               N_DIRECTOR)
    h_a, al_a, mx_a = _project(x_actor, Wp_actor, bp_actor, A_a, N_ACTOR)

    s_md, d_md = _pad_edges(src_md, dst_md, 53248)
    s_dm, d_dm = _pad_edges(src_dm, dst_dm, 53248)
    s_ma, d_ma = _pad_edges(src_ma, dst_ma, 151552)
    s_am, d_am = _pad_edges(src_am, dst_am, 151552)

    out_md = _edge_md(_headmajor(h_m, 4, N_MOVIE), al_m[:, 0:16],
                      al_d[:, 0:16], s_md, d_md, _mshift(mx_m, 0, mx_d, 0))
    out_dm = _edge_dm(_headmajor(h_d, 1, N_DIRECTOR), al_d[:, 8:24],
                      al_m[:, 16:32], s_dm, d_dm, _mshift(mx_d, 8, mx_m, 16))
    out_ma = _edge_ma(_headmajor(h_m, 2, N_MOVIE), al_m[:, 8:24],
                      al_a[:, 0:16], s_ma, d_ma, _mshift(mx_m, 8, mx_a, 0))
    out_am = _edge_am(_headmajor(h_a, 1, N_ACTOR), al_a[:, 8:24],
                      al_m[:, 24:40], s_am, d_am, _mshift(mx_a, 8, mx_m, 24))

    o_dm = _regroup(out_dm, N_MOVIE)
    o_am = _regroup(out_am, N_MOVIE)
    director_out = _regroup(out_md, N_DIRECTOR)
    actor_out = _regroup(out_ma, N_ACTOR)

    t = _tansum(o_dm, o_am, kW, kb)
    movie_out, lsm = _combine(o_dm, o_am, t, q, lin_W, lin_b)
    return (lsm[:, :OUT], movie_out, director_out, actor_out)
